# baseline (speedup 1.0000x reference)
import jax
import jax.numpy as jnp
from jax.experimental import pallas as pl
from jax.experimental.pallas import tpu as pltpu
from jax.experimental.pallas import tpu_sc as plsc

B = 4096
D = 384
N = 100000
S = 256
K = 3
W_STRUCT = 0.3
REL_THRESH = 0.1

BK = 1024
BQ = 1024
NQ = B // BQ
NK = (N + BK - 1) // BK
NG = BK // 128
NKP = NK * BK
NEG = -1.0e30
PACK = (BQ // 128, 128)


_BN = 2048


def _norm_kernel(k_ref, o_ref):
    i = pl.program_id(0)
    kb = k_ref[...]
    nrm = jnp.sqrt(jnp.sum(kb * kb, axis=1, keepdims=True))
    kn = (kb / (nrm + 1e-12)).astype(jnp.bfloat16)
    row = jax.lax.broadcasted_iota(jnp.int32, (_BN, 1), 0) + i * _BN
    o_ref[...] = jnp.where(row < N, kn, jnp.bfloat16(0))


def _normalize_keys(te):
    return pl.pallas_call(
        _norm_kernel,
        grid=(NKP // _BN,),
        in_specs=[pl.BlockSpec((_BN, D), lambda i: (i, 0))],
        out_specs=pl.BlockSpec((_BN, D), lambda i: (i, 0)),
        out_shape=jax.ShapeDtypeStruct((NKP, D), jnp.bfloat16),
    )(te)


def _merge_top3(av, ai, bv, bi):
    out_v, out_i = [], []
    c = av[0] >= bv[0]
    out_v.append(jnp.where(c, av[0], bv[0]))
    out_i.append(jnp.where(c, ai[0], bi[0]))
    a1v = [jnp.where(c, av[1], av[0]), jnp.where(c, av[2], av[1])]
    a1i = [jnp.where(c, ai[1], ai[0]), jnp.where(c, ai[2], ai[1])]
    b1v = [jnp.where(c, bv[0], bv[1]), jnp.where(c, bv[1], bv[2])]
    b1i = [jnp.where(c, bi[0], bi[1]), jnp.where(c, bi[1], bi[2])]
    c = a1v[0] >= b1v[0]
    out_v.append(jnp.where(c, a1v[0], b1v[0]))
    out_i.append(jnp.where(c, a1i[0], b1i[0]))
    a2v = jnp.where(c, a1v[1], a1v[0])
    a2i = jnp.where(c, a1i[1], a1i[0])
    b2v = jnp.where(c, b1v[0], b1v[1])
    b2i = jnp.where(c, b1i[0], b1i[1])
    c = a2v >= b2v
    out_v.append(jnp.where(c, a2v, b2v))
    out_i.append(jnp.where(c, a2i, b2i))
    return out_v, out_i


def _pack(x):
    return x.reshape(PACK)


def _topk_kernel(q_ref, kn_ref, w_ref, idx_ref, qn_ref, sb_ref,
                 t0_ref, t1_ref, t2_ref, g0_ref, g1_ref, g2_ref):
    kblk = pl.program_id(1)

    @pl.when(kblk == 0)
    def _():
        q = q_ref[...]
        qn = q / (jnp.sqrt(jnp.sum(q * q, axis=1, keepdims=True)) + 1e-12)
        qn_ref[...] = qn.astype(jnp.bfloat16)
        t0_ref[...] = jnp.full((BQ, 128), NEG, jnp.float32)
        t1_ref[...] = jnp.full((BQ, 128), NEG, jnp.float32)
        t2_ref[...] = jnp.full((BQ, 128), NEG, jnp.float32)
        g0_ref[...] = jnp.zeros((BQ, 128), jnp.int32)
        g1_ref[...] = jnp.zeros((BQ, 128), jnp.int32)
        g2_ref[...] = jnp.zeros((BQ, 128), jnp.int32)

    par = jax.lax.rem(kblk, 2)

    @pl.when(kblk < NK)
    def _():
        kn = kn_ref[...]
        sb_ref[par] = jax.lax.dot_general(
            qn_ref[...], kn, (((1,), (1,)), ((), ())),
            preferred_element_type=jnp.float32)

    liota = jax.lax.broadcasted_iota(jnp.int32, (BQ, 128), 1)

    @pl.when(kblk > 0)
    def _():
        s = sb_ref[1 - par]
        t0, t1, t2 = t0_ref[...], t1_ref[...], t2_ref[...]
        g0, g1, g2 = g0_ref[...], g1_ref[...], g2_ref[...]
        for g in range(NG):
            x = s[:, g * 128:(g + 1) * 128]
            gid = (kblk - 1) * NG + g
            c0 = x > t0
            nx = jnp.where(c0, t0, x)
            ng = jnp.where(c0, g0, gid)
            t0 = jnp.where(c0, x, t0)
            g0 = jnp.where(c0, gid, g0)
            c1 = nx > t1
            nx2 = jnp.where(c1, t1, nx)
            ng2 = jnp.where(c1, g1, ng)
            t1 = jnp.where(c1, nx, t1)
            g1 = jnp.where(c1, ng, g1)
            c2 = nx2 > t2
            t2 = jnp.where(c2, nx2, t2)
            g2 = jnp.where(c2, ng2, g2)
        t0_ref[...], t1_ref[...], t2_ref[...] = t0, t1, t2
        g0_ref[...], g1_ref[...], g2_ref[...] = g0, g1, g2

    @pl.when(kblk == NK)
    def _():
        t0, t1, t2 = t0_ref[...], t1_ref[...], t2_ref[...]
        g0, g1, g2 = g0_ref[...], g1_ref[...], g2_ref[...]
        cand = jnp.concatenate([t0, t1, t2], axis=1)
        cidx = (jnp.concatenate([g0, g1, g2], axis=1) * 128
                + jnp.concatenate([liota, liota, liota], axis=1))
        ei = jax.lax.broadcasted_iota(jnp.int32, (BQ, K * 128), 1)
        vs, is_ = [], []
        for j in range(K):
            m = jnp.max(cand, axis=1, keepdims=True)
            p = jnp.min(jnp.where(cand == m, ei, K * 128), axis=1,
                        keepdims=True)
            onp = ei == p
            ii = jnp.max(jnp.where(onp, cidx, -1), axis=1, keepdims=True)
            ii = jnp.minimum(ii, N - 1)
            vs.append(_pack(m))
            is_.append(_pack(ii))
            if j < K - 1:
                cand = jnp.where(onp, NEG, cand)
        ms = [jnp.where(v > REL_THRESH, v, 0.0) for v in vs]
        ssum = ms[0] + ms[1] + ms[2] + 1e-8
        for j in range(K):
            w_ref[j, :, :] = ms[j] / ssum
            idx_ref[j, :, :] = is_[j]


def _topk(q, kn16):
    return pl.pallas_call(
        _topk_kernel,
        grid=(NQ, NK + 1),
        in_specs=[
            pl.BlockSpec((BQ, D), lambda i, k: (i, 0)),
            pl.BlockSpec((BK, D), lambda i, k: (jnp.minimum(k, NK - 1), 0)),
        ],
        out_specs=[
            pl.BlockSpec((K, PACK[0], 128), lambda i, k: (0, i, 0)),
            pl.BlockSpec((K, PACK[0], 128), lambda i, k: (0, i, 0)),
        ],
        out_shape=[
            jax.ShapeDtypeStruct((K, B // 128, 128), jnp.float32),
            jax.ShapeDtypeStruct((K, B // 128, 128), jnp.int32),
        ],
        scratch_shapes=[
            pltpu.VMEM((BQ, D), jnp.bfloat16),
            pltpu.VMEM((2, BQ, BK), jnp.float32),
            pltpu.VMEM((BQ, 128), jnp.float32),
            pltpu.VMEM((BQ, 128), jnp.float32),
            pltpu.VMEM((BQ, 128), jnp.float32),
            pltpu.VMEM((BQ, 128), jnp.int32),
            pltpu.VMEM((BQ, 128), jnp.int32),
            pltpu.VMEM((BQ, 128), jnp.int32),
        ],
    )(q, kn16)


_GATHER_WIN = 128


def _sc_gather(struct, idx_flat):
    n_idx = B * K

    @pl.kernel(
        out_type=jax.ShapeDtypeStruct((n_idx, S), jnp.float32),
        mesh=plsc.VectorSubcoreMesh(core_axis_name="core",
                                    subcore_axis_name="subcore"),
    )
    def _gather(x_hbm, i_hbm, o_hbm):
        def body(i_vmem, o_vmem):
            pltpu.sync_copy(x_hbm.at[i_vmem.at[0]], o_vmem)

        pltpu.emit_pipeline(
            body,
            grid=(n_idx // _GATHER_WIN,),
            in_specs=[pl.BlockSpec((1, _GATHER_WIN), lambda i: (0, i))],
            out_specs=[pl.BlockSpec((_GATHER_WIN, S), lambda i: (i, 0))],
            core_axis_name="subcore",
            dimension_semantics=(pltpu.PARALLEL,),
        )(i_hbm, o_hbm)

    return _gather(struct, idx_flat)


def _combine_kernel(w_ref, r0_ref, r1_ref, r2_ref, o_ref):
    w = w_ref[...]
    acc = (w[:, 0:1] * r0_ref[...] + w[:, 1:2] * r1_ref[...]
           + w[:, 2:3] * r2_ref[...])
    o_ref[...] = W_STRUCT * acc


def _combine(w, r0, r1, r2):
    bq = 1024
    return pl.pallas_call(
        _combine_kernel,
        grid=(B // bq,),
        in_specs=[
            pl.BlockSpec((bq, K), lambda i: (i, 0)),
            pl.BlockSpec((bq, S), lambda i: (i, 0)),
            pl.BlockSpec((bq, S), lambda i: (i, 0)),
            pl.BlockSpec((bq, S), lambda i: (i, 0)),
        ],
        out_specs=pl.BlockSpec((bq, S), lambda i: (i, 0)),
        out_shape=jax.ShapeDtypeStruct((B, S), jnp.float32),
    )(w, r0, r1, r2)


def kernel(sentence_embeddings, text_embeddings, structural_embeddings):
    kn16 = _normalize_keys(text_embeddings)
    wp, idxp = _topk(sentence_embeddings, kn16)
    idx_flat = idxp.reshape(1, K * B)
    retrieved = _sc_gather(structural_embeddings, idx_flat)
    r = retrieved.reshape(K, B, S)
    w = jnp.concatenate([wp[j].reshape(B, 1) for j in range(K)], axis=1)
    struct_ctx = _combine(w, r[0], r[1], r[2])
    return jnp.concatenate([sentence_embeddings, struct_ctx], axis=-1)


# speedup vs baseline: 1.9529x; 1.9529x over previous
import jax
import jax.numpy as jnp
from jax.experimental import pallas as pl
from jax.experimental.pallas import tpu as pltpu
from jax.experimental.pallas import tpu_sc as plsc

B = 4096
D = 384
N = 100000
S = 256
K = 3
W_STRUCT = 0.3
REL_THRESH = 0.1

BK = 1024
BQ = 1024
NQ = B // BQ
NK = (N + BK - 1) // BK
NG = BK // 128
NKP = NK * BK
NEG = -1.0e30
PACK = (BQ // 128, 128)


_BN = 2048


def _norm_kernel(k_ref, o_ref):
    i = pl.program_id(0)
    kb = k_ref[...]
    nrm = jnp.sqrt(jnp.sum(kb * kb, axis=1, keepdims=True))
    kn = (kb / (nrm + 1e-12)).astype(jnp.bfloat16)
    row = jax.lax.broadcasted_iota(jnp.int32, (_BN, 1), 0) + i * _BN
    o_ref[...] = jnp.where(row < N, kn, jnp.bfloat16(0))


def _normalize_keys(te):
    return pl.pallas_call(
        _norm_kernel,
        grid=(NKP // _BN,),
        in_specs=[pl.BlockSpec((_BN, D), lambda i: (i, 0))],
        out_specs=pl.BlockSpec((_BN, D), lambda i: (i, 0)),
        out_shape=jax.ShapeDtypeStruct((NKP, D), jnp.bfloat16),
    )(te)


def _merge_top3(av, ai, bv, bi):
    out_v, out_i = [], []
    c = av[0] >= bv[0]
    out_v.append(jnp.where(c, av[0], bv[0]))
    out_i.append(jnp.where(c, ai[0], bi[0]))
    a1v = [jnp.where(c, av[1], av[0]), jnp.where(c, av[2], av[1])]
    a1i = [jnp.where(c, ai[1], ai[0]), jnp.where(c, ai[2], ai[1])]
    b1v = [jnp.where(c, bv[0], bv[1]), jnp.where(c, bv[1], bv[2])]
    b1i = [jnp.where(c, bi[0], bi[1]), jnp.where(c, bi[1], bi[2])]
    c = a1v[0] >= b1v[0]
    out_v.append(jnp.where(c, a1v[0], b1v[0]))
    out_i.append(jnp.where(c, a1i[0], b1i[0]))
    a2v = jnp.where(c, a1v[1], a1v[0])
    a2i = jnp.where(c, a1i[1], a1i[0])
    b2v = jnp.where(c, b1v[0], b1v[1])
    b2i = jnp.where(c, b1i[0], b1i[1])
    c = a2v >= b2v
    out_v.append(jnp.where(c, a2v, b2v))
    out_i.append(jnp.where(c, a2i, b2i))
    return out_v, out_i


def _pack(x):
    return x.reshape(PACK)


def _topk_kernel(q_ref, kn_ref, w_ref, idx_ref, qn_ref,
                 t0_ref, t1_ref, t2_ref, g0_ref, g1_ref, g2_ref):
    kblk = pl.program_id(1)

    @pl.when(kblk == 0)
    def _():
        q = q_ref[...]
        qn = q / (jnp.sqrt(jnp.sum(q * q, axis=1, keepdims=True)) + 1e-12)
        qn_ref[...] = qn.astype(jnp.bfloat16)
        t0_ref[...] = jnp.full((BQ, 128), NEG, jnp.float32)
        t1_ref[...] = jnp.full((BQ, 128), NEG, jnp.float32)
        t2_ref[...] = jnp.full((BQ, 128), NEG, jnp.float32)
        g0_ref[...] = jnp.zeros((BQ, 128), jnp.int32)
        g1_ref[...] = jnp.zeros((BQ, 128), jnp.int32)
        g2_ref[...] = jnp.zeros((BQ, 128), jnp.int32)

    kn = kn_ref[...]
    s = jax.lax.dot_general(qn_ref[...], kn, (((1,), (1,)), ((), ())),
                            preferred_element_type=jnp.float32)

    liota = jax.lax.broadcasted_iota(jnp.int32, (BQ, 128), 1)

    t0, t1, t2 = t0_ref[...], t1_ref[...], t2_ref[...]
    g0, g1, g2 = g0_ref[...], g1_ref[...], g2_ref[...]
    for g in range(NG):
        x = s[:, g * 128:(g + 1) * 128]
        gid = kblk * NG + g
        c0 = x > t0
        nx = jnp.where(c0, t0, x)
        ng = jnp.where(c0, g0, gid)
        t0 = jnp.where(c0, x, t0)
        g0 = jnp.where(c0, gid, g0)
        c1 = nx > t1
        nx2 = jnp.where(c1, t1, nx)
        ng2 = jnp.where(c1, g1, ng)
        t1 = jnp.where(c1, nx, t1)
        g1 = jnp.where(c1, ng, g1)
        c2 = nx2 > t2
        t2 = jnp.where(c2, nx2, t2)
        g2 = jnp.where(c2, ng2, g2)
    t0_ref[...], t1_ref[...], t2_ref[...] = t0, t1, t2
    g0_ref[...], g1_ref[...], g2_ref[...] = g0, g1, g2

    @pl.when(kblk == NK - 1)
    def _():
        t0, t1, t2 = t0_ref[...], t1_ref[...], t2_ref[...]
        g0, g1, g2 = g0_ref[...], g1_ref[...], g2_ref[...]
        cand = jnp.concatenate([t0, t1, t2], axis=1)
        cidx = (jnp.concatenate([g0, g1, g2], axis=1) * 128
                + jnp.concatenate([liota, liota, liota], axis=1))
        ei = jax.lax.broadcasted_iota(jnp.int32, (BQ, K * 128), 1)
        vs, is_ = [], []
        for j in range(K):
            m = jnp.max(cand, axis=1, keepdims=True)
            p = jnp.min(jnp.where(cand == m, ei, K * 128), axis=1,
                        keepdims=True)
            onp = ei == p
            ii = jnp.max(jnp.where(onp, cidx, -1), axis=1, keepdims=True)
            ii = jnp.minimum(ii, N - 1)
            vs.append(_pack(m))
            is_.append(_pack(ii))
            if j < K - 1:
                cand = jnp.where(onp, NEG, cand)
        ms = [jnp.where(v > REL_THRESH, v, 0.0) for v in vs]
        ssum = ms[0] + ms[1] + ms[2] + 1e-8
        for j in range(K):
            w_ref[j, :, :] = ms[j] / ssum
            idx_ref[j, :, :] = is_[j]


def _topk(q, kn16):
    return pl.pallas_call(
        _topk_kernel,
        grid=(NQ, NK),
        in_specs=[
            pl.BlockSpec((BQ, D), lambda i, k: (i, 0)),
            pl.BlockSpec((BK, D), lambda i, k: (k, 0)),
        ],
        out_specs=[
            pl.BlockSpec((K, PACK[0], 128), lambda i, k: (0, i, 0)),
            pl.BlockSpec((K, PACK[0], 128), lambda i, k: (0, i, 0)),
        ],
        out_shape=[
            jax.ShapeDtypeStruct((K, B // 128, 128), jnp.float32),
            jax.ShapeDtypeStruct((K, B // 128, 128), jnp.int32),
        ],
        scratch_shapes=[
            pltpu.VMEM((BQ, D), jnp.bfloat16),
            pltpu.VMEM((BQ, 128), jnp.float32),
            pltpu.VMEM((BQ, 128), jnp.float32),
            pltpu.VMEM((BQ, 128), jnp.float32),
            pltpu.VMEM((BQ, 128), jnp.int32),
            pltpu.VMEM((BQ, 128), jnp.int32),
            pltpu.VMEM((BQ, 128), jnp.int32),
        ],
    )(q, kn16)


_GATHER_WIN = 128


def _sc_gather(struct, idx_flat):
    n_idx = B * K

    @pl.kernel(
        out_type=jax.ShapeDtypeStruct((n_idx, S), jnp.float32),
        mesh=plsc.VectorSubcoreMesh(core_axis_name="core",
                                    subcore_axis_name="subcore"),
    )
    def _gather(x_hbm, i_hbm, o_hbm):
        def body(i_vmem, o_vmem):
            pltpu.sync_copy(x_hbm.at[i_vmem.at[0]], o_vmem)

        pltpu.emit_pipeline(
            body,
            grid=(n_idx // _GATHER_WIN,),
            in_specs=[pl.BlockSpec((1, _GATHER_WIN), lambda i: (0, i))],
            out_specs=[pl.BlockSpec((_GATHER_WIN, S), lambda i: (i, 0))],
            core_axis_name="subcore",
            dimension_semantics=(pltpu.PARALLEL,),
        )(i_hbm, o_hbm)

    return _gather(struct, idx_flat)


def _combine_kernel(w_ref, r0_ref, r1_ref, r2_ref, o_ref):
    w = w_ref[...]
    acc = (w[:, 0:1] * r0_ref[...] + w[:, 1:2] * r1_ref[...]
           + w[:, 2:3] * r2_ref[...])
    o_ref[...] = W_STRUCT * acc


def _combine(w, r0, r1, r2):
    bq = 1024
    return pl.pallas_call(
        _combine_kernel,
        grid=(B // bq,),
        in_specs=[
            pl.BlockSpec((bq, K), lambda i: (i, 0)),
            pl.BlockSpec((bq, S), lambda i: (i, 0)),
            pl.BlockSpec((bq, S), lambda i: (i, 0)),
            pl.BlockSpec((bq, S), lambda i: (i, 0)),
        ],
        out_specs=pl.BlockSpec((bq, S), lambda i: (i, 0)),
        out_shape=jax.ShapeDtypeStruct((B, S), jnp.float32),
    )(w, r0, r1, r2)


def kernel(sentence_embeddings, text_embeddings, structural_embeddings):
    kn16 = _normalize_keys(text_embeddings)
    wp, idxp = _topk(sentence_embeddings, kn16)
    idx_flat = idxp.reshape(1, K * B)
    retrieved = _sc_gather(structural_embeddings, idx_flat)
    r = retrieved.reshape(K, B, S)
    w = jnp.concatenate([wp[j].reshape(B, 1) for j in range(K)], axis=1)
    struct_ctx = _combine(w, r[0], r[1], r[2])
    return jnp.concatenate([sentence_embeddings, struct_ctx], axis=-1)


# speedup vs baseline: 1.9788x; 1.0133x over previous
import jax
import jax.numpy as jnp
from jax.experimental import pallas as pl
from jax.experimental.pallas import tpu as pltpu
from jax.experimental.pallas import tpu_sc as plsc

B = 4096
D = 384
N = 100000
S = 256
K = 3
W_STRUCT = 0.3
REL_THRESH = 0.1

BK = 1024
BQ = 1024
NQ = B // BQ
NK = (N + BK - 1) // BK
NG = BK // 128
NKP = NK * BK
NEG = -1.0e30
PACK = (BQ // 128, 128)


_BN = 2048


def _norm_kernel(k_ref, o_ref):
    i = pl.program_id(0)
    kb = k_ref[...]
    nrm = jnp.sqrt(jnp.sum(kb * kb, axis=1, keepdims=True))
    kn = (kb / (nrm + 1e-12)).astype(jnp.bfloat16)
    row = jax.lax.broadcasted_iota(jnp.int32, (_BN, 1), 0) + i * _BN
    kn = jnp.where(row < N, kn, jnp.bfloat16(0))
    o_ref[...] = kn.T


def _normalize_keys(te):
    return pl.pallas_call(
        _norm_kernel,
        grid=(NKP // _BN,),
        in_specs=[pl.BlockSpec((_BN, D), lambda i: (i, 0))],
        out_specs=pl.BlockSpec((D, _BN), lambda i: (0, i)),
        out_shape=jax.ShapeDtypeStruct((D, NKP), jnp.bfloat16),
    )(te)


def _merge_top3(av, ai, bv, bi):
    out_v, out_i = [], []
    c = av[0] >= bv[0]
    out_v.append(jnp.where(c, av[0], bv[0]))
    out_i.append(jnp.where(c, ai[0], bi[0]))
    a1v = [jnp.where(c, av[1], av[0]), jnp.where(c, av[2], av[1])]
    a1i = [jnp.where(c, ai[1], ai[0]), jnp.where(c, ai[2], ai[1])]
    b1v = [jnp.where(c, bv[0], bv[1]), jnp.where(c, bv[1], bv[2])]
    b1i = [jnp.where(c, bi[0], bi[1]), jnp.where(c, bi[1], bi[2])]
    c = a1v[0] >= b1v[0]
    out_v.append(jnp.where(c, a1v[0], b1v[0]))
    out_i.append(jnp.where(c, a1i[0], b1i[0]))
    a2v = jnp.where(c, a1v[1], a1v[0])
    a2i = jnp.where(c, a1i[1], a1i[0])
    b2v = jnp.where(c, b1v[0], b1v[1])
    b2i = jnp.where(c, b1i[0], b1i[1])
    c = a2v >= b2v
    out_v.append(jnp.where(c, a2v, b2v))
    out_i.append(jnp.where(c, a2i, b2i))
    return out_v, out_i


def _pack(x):
    return x.reshape(PACK)


def _topk_kernel(q_ref, kn_ref, w_ref, idx_ref, qn_ref,
                 t0_ref, t1_ref, t2_ref, g0_ref, g1_ref, g2_ref):
    kblk = pl.program_id(1)

    @pl.when(kblk == 0)
    def _():
        q = q_ref[...]
        qn = q / (jnp.sqrt(jnp.sum(q * q, axis=1, keepdims=True)) + 1e-12)
        qn_ref[...] = qn.astype(jnp.bfloat16)
        t0_ref[...] = jnp.full((BQ, 128), NEG, jnp.float32)
        t1_ref[...] = jnp.full((BQ, 128), NEG, jnp.float32)
        t2_ref[...] = jnp.full((BQ, 128), NEG, jnp.float32)
        g0_ref[...] = jnp.zeros((BQ, 128), jnp.int32)
        g1_ref[...] = jnp.zeros((BQ, 128), jnp.int32)
        g2_ref[...] = jnp.zeros((BQ, 128), jnp.int32)

    kn = kn_ref[...]
    s = jax.lax.dot_general(qn_ref[...], kn, (((1,), (0,)), ((), ())),
                            preferred_element_type=jnp.float32)

    liota = jax.lax.broadcasted_iota(jnp.int32, (BQ, 128), 1)

    t0, t1, t2 = t0_ref[...], t1_ref[...], t2_ref[...]
    g0, g1, g2 = g0_ref[...], g1_ref[...], g2_ref[...]
    for g in range(NG):
        x = s[:, g * 128:(g + 1) * 128]
        gid = kblk * NG + g
        c0 = x > t0
        nx = jnp.where(c0, t0, x)
        ng = jnp.where(c0, g0, gid)
        t0 = jnp.where(c0, x, t0)
        g0 = jnp.where(c0, gid, g0)
        c1 = nx > t1
        nx2 = jnp.where(c1, t1, nx)
        ng2 = jnp.where(c1, g1, ng)
        t1 = jnp.where(c1, nx, t1)
        g1 = jnp.where(c1, ng, g1)
        c2 = nx2 > t2
        t2 = jnp.where(c2, nx2, t2)
        g2 = jnp.where(c2, ng2, g2)
    t0_ref[...], t1_ref[...], t2_ref[...] = t0, t1, t2
    g0_ref[...], g1_ref[...], g2_ref[...] = g0, g1, g2

    @pl.when(kblk == NK - 1)
    def _():
        t0, t1, t2 = t0_ref[...], t1_ref[...], t2_ref[...]
        g0, g1, g2 = g0_ref[...], g1_ref[...], g2_ref[...]
        cand = jnp.concatenate([t0, t1, t2], axis=1)
        cidx = (jnp.concatenate([g0, g1, g2], axis=1) * 128
                + jnp.concatenate([liota, liota, liota], axis=1))
        ei = jax.lax.broadcasted_iota(jnp.int32, (BQ, K * 128), 1)
        vs, is_ = [], []
        for j in range(K):
            m = jnp.max(cand, axis=1, keepdims=True)
            p = jnp.min(jnp.where(cand == m, ei, K * 128), axis=1,
                        keepdims=True)
            onp = ei == p
            ii = jnp.max(jnp.where(onp, cidx, -1), axis=1, keepdims=True)
            ii = jnp.minimum(ii, N - 1)
            vs.append(_pack(m))
            is_.append(_pack(ii))
            if j < K - 1:
                cand = jnp.where(onp, NEG, cand)
        ms = [jnp.where(v > REL_THRESH, v, 0.0) for v in vs]
        ssum = ms[0] + ms[1] + ms[2] + 1e-8
        for j in range(K):
            w_ref[j, :, :] = ms[j] / ssum
            idx_ref[j, :, :] = is_[j]


def _topk(q, kn16):
    return pl.pallas_call(
        _topk_kernel,
        grid=(NQ, NK),
        in_specs=[
            pl.BlockSpec((BQ, D), lambda i, k: (i, 0)),
            pl.BlockSpec((D, BK), lambda i, k: (0, k)),
        ],
        out_specs=[
            pl.BlockSpec((K, PACK[0], 128), lambda i, k: (0, i, 0)),
            pl.BlockSpec((K, PACK[0], 128), lambda i, k: (0, i, 0)),
        ],
        out_shape=[
            jax.ShapeDtypeStruct((K, B // 128, 128), jnp.float32),
            jax.ShapeDtypeStruct((K, B // 128, 128), jnp.int32),
        ],
        scratch_shapes=[
            pltpu.VMEM((BQ, D), jnp.bfloat16),
            pltpu.VMEM((BQ, 128), jnp.float32),
            pltpu.VMEM((BQ, 128), jnp.float32),
            pltpu.VMEM((BQ, 128), jnp.float32),
            pltpu.VMEM((BQ, 128), jnp.int32),
            pltpu.VMEM((BQ, 128), jnp.int32),
            pltpu.VMEM((BQ, 128), jnp.int32),
        ],
    )(q, kn16)


_GATHER_WIN = 128


def _sc_gather(struct, idx_flat):
    n_idx = B * K

    @pl.kernel(
        out_type=jax.ShapeDtypeStruct((n_idx, S), jnp.float32),
        mesh=plsc.VectorSubcoreMesh(core_axis_name="core",
                                    subcore_axis_name="subcore"),
    )
    def _gather(x_hbm, i_hbm, o_hbm):
        def body(i_vmem, o_vmem):
            pltpu.sync_copy(x_hbm.at[i_vmem.at[0]], o_vmem)

        pltpu.emit_pipeline(
            body,
            grid=(n_idx // _GATHER_WIN,),
            in_specs=[pl.BlockSpec((1, _GATHER_WIN), lambda i: (0, i))],
            out_specs=[pl.BlockSpec((_GATHER_WIN, S), lambda i: (i, 0))],
            core_axis_name="subcore",
            dimension_semantics=(pltpu.PARALLEL,),
        )(i_hbm, o_hbm)

    return _gather(struct, idx_flat)


def _combine_kernel(w_ref, r0_ref, r1_ref, r2_ref, o_ref):
    w = w_ref[...]
    acc = (w[:, 0:1] * r0_ref[...] + w[:, 1:2] * r1_ref[...]
           + w[:, 2:3] * r2_ref[...])
    o_ref[...] = W_STRUCT * acc


def _combine(w, r0, r1, r2):
    bq = 1024
    return pl.pallas_call(
        _combine_kernel,
        grid=(B // bq,),
        in_specs=[
            pl.BlockSpec((bq, K), lambda i: (i, 0)),
            pl.BlockSpec((bq, S), lambda i: (i, 0)),
            pl.BlockSpec((bq, S), lambda i: (i, 0)),
            pl.BlockSpec((bq, S), lambda i: (i, 0)),
        ],
        out_specs=pl.BlockSpec((bq, S), lambda i: (i, 0)),
        out_shape=jax.ShapeDtypeStruct((B, S), jnp.float32),
    )(w, r0, r1, r2)


def kernel(sentence_embeddings, text_embeddings, structural_embeddings):
    kn16 = _normalize_keys(text_embeddings)
    wp, idxp = _topk(sentence_embeddings, kn16)
    idx_flat = idxp.reshape(1, K * B)
    retrieved = _sc_gather(structural_embeddings, idx_flat)
    r = retrieved.reshape(K, B, S)
    w = jnp.concatenate([wp[j].reshape(B, 1) for j in range(K)], axis=1)
    struct_ctx = _combine(w, r[0], r[1], r[2])
    return jnp.concatenate([sentence_embeddings, struct_ctx], axis=-1)


# speedup vs baseline: 2.0208x; 1.0212x over previous
import jax
import jax.numpy as jnp
from jax.experimental import pallas as pl
from jax.experimental.pallas import tpu as pltpu
from jax.experimental.pallas import tpu_sc as plsc

B = 4096
D = 384
N = 100000
S = 256
K = 3
W_STRUCT = 0.3
REL_THRESH = 0.1

BK = 1024
BQ = 2048
NQ = B // BQ
NK = (N + BK - 1) // BK
NG = BK // 128
NKP = NK * BK
NEG = -1.0e30
PACK = (BQ // 128, 128)


_BN = 2048


def _norm_kernel(k_ref, o_ref):
    i = pl.program_id(0)
    kb = k_ref[...]
    nrm = jnp.sqrt(jnp.sum(kb * kb, axis=1, keepdims=True))
    kn = (kb / (nrm + 1e-12)).astype(jnp.bfloat16)
    row = jax.lax.broadcasted_iota(jnp.int32, (_BN, 1), 0) + i * _BN
    kn = jnp.where(row < N, kn, jnp.bfloat16(0))
    o_ref[...] = kn.T


def _normalize_keys(te):
    return pl.pallas_call(
        _norm_kernel,
        grid=(NKP // _BN,),
        in_specs=[pl.BlockSpec((_BN, D), lambda i: (i, 0))],
        out_specs=pl.BlockSpec((D, _BN), lambda i: (0, i)),
        out_shape=jax.ShapeDtypeStruct((D, NKP), jnp.bfloat16),
    )(te)


def _merge_top3(av, ai, bv, bi):
    out_v, out_i = [], []
    c = av[0] >= bv[0]
    out_v.append(jnp.where(c, av[0], bv[0]))
    out_i.append(jnp.where(c, ai[0], bi[0]))
    a1v = [jnp.where(c, av[1], av[0]), jnp.where(c, av[2], av[1])]
    a1i = [jnp.where(c, ai[1], ai[0]), jnp.where(c, ai[2], ai[1])]
    b1v = [jnp.where(c, bv[0], bv[1]), jnp.where(c, bv[1], bv[2])]
    b1i = [jnp.where(c, bi[0], bi[1]), jnp.where(c, bi[1], bi[2])]
    c = a1v[0] >= b1v[0]
    out_v.append(jnp.where(c, a1v[0], b1v[0]))
    out_i.append(jnp.where(c, a1i[0], b1i[0]))
    a2v = jnp.where(c, a1v[1], a1v[0])
    a2i = jnp.where(c, a1i[1], a1i[0])
    b2v = jnp.where(c, b1v[0], b1v[1])
    b2i = jnp.where(c, b1i[0], b1i[1])
    c = a2v >= b2v
    out_v.append(jnp.where(c, a2v, b2v))
    out_i.append(jnp.where(c, a2i, b2i))
    return out_v, out_i


def _pack(x):
    return x.reshape(PACK)


def _topk_kernel(q_ref, kn_ref, w_ref, idx_ref, qn_ref,
                 t0_ref, t1_ref, t2_ref, g0_ref, g1_ref, g2_ref):
    kblk = pl.program_id(1)

    @pl.when(kblk == 0)
    def _():
        q = q_ref[...]
        qn = q / (jnp.sqrt(jnp.sum(q * q, axis=1, keepdims=True)) + 1e-12)
        qn_ref[...] = qn.astype(jnp.bfloat16)
        t0_ref[...] = jnp.full((BQ, 128), NEG, jnp.float32)
        t1_ref[...] = jnp.full((BQ, 128), NEG, jnp.float32)
        t2_ref[...] = jnp.full((BQ, 128), NEG, jnp.float32)
        g0_ref[...] = jnp.zeros((BQ, 128), jnp.int32)
        g1_ref[...] = jnp.zeros((BQ, 128), jnp.int32)
        g2_ref[...] = jnp.zeros((BQ, 128), jnp.int32)

    kn = kn_ref[...]
    s = jax.lax.dot_general(qn_ref[...], kn, (((1,), (0,)), ((), ())),
                            preferred_element_type=jnp.float32)

    liota = jax.lax.broadcasted_iota(jnp.int32, (BQ, 128), 1)

    t0, t1, t2 = t0_ref[...], t1_ref[...], t2_ref[...]
    g0, g1, g2 = g0_ref[...], g1_ref[...], g2_ref[...]
    for g in range(NG):
        x = s[:, g * 128:(g + 1) * 128]
        gid = kblk * NG + g
        c0 = x > t0
        nx = jnp.where(c0, t0, x)
        ng = jnp.where(c0, g0, gid)
        t0 = jnp.where(c0, x, t0)
        g0 = jnp.where(c0, gid, g0)
        c1 = nx > t1
        nx2 = jnp.where(c1, t1, nx)
        ng2 = jnp.where(c1, g1, ng)
        t1 = jnp.where(c1, nx, t1)
        g1 = jnp.where(c1, ng, g1)
        c2 = nx2 > t2
        t2 = jnp.where(c2, nx2, t2)
        g2 = jnp.where(c2, ng2, g2)
    t0_ref[...], t1_ref[...], t2_ref[...] = t0, t1, t2
    g0_ref[...], g1_ref[...], g2_ref[...] = g0, g1, g2

    @pl.when(kblk == NK - 1)
    def _():
        t0, t1, t2 = t0_ref[...], t1_ref[...], t2_ref[...]
        g0, g1, g2 = g0_ref[...], g1_ref[...], g2_ref[...]
        cand = jnp.concatenate([t0, t1, t2], axis=1)
        cidx = (jnp.concatenate([g0, g1, g2], axis=1) * 128
                + jnp.concatenate([liota, liota, liota], axis=1))
        ei = jax.lax.broadcasted_iota(jnp.int32, (BQ, K * 128), 1)
        vs, is_ = [], []
        for j in range(K):
            m = jnp.max(cand, axis=1, keepdims=True)
            p = jnp.min(jnp.where(cand == m, ei, K * 128), axis=1,
                        keepdims=True)
            onp = ei == p
            ii = jnp.max(jnp.where(onp, cidx, -1), axis=1, keepdims=True)
            ii = jnp.minimum(ii, N - 1)
            vs.append(_pack(m))
            is_.append(_pack(ii))
            if j < K - 1:
                cand = jnp.where(onp, NEG, cand)
        ms = [jnp.where(v > REL_THRESH, v, 0.0) for v in vs]
        ssum = ms[0] + ms[1] + ms[2] + 1e-8
        for j in range(K):
            w_ref[j, :, :] = ms[j] / ssum
            idx_ref[j, :, :] = is_[j]


def _topk(q, kn16):
    return pl.pallas_call(
        _topk_kernel,
        grid=(NQ, NK),
        in_specs=[
            pl.BlockSpec((BQ, D), lambda i, k: (i, 0)),
            pl.BlockSpec((D, BK), lambda i, k: (0, k)),
        ],
        out_specs=[
            pl.BlockSpec((K, PACK[0], 128), lambda i, k: (0, i, 0)),
            pl.BlockSpec((K, PACK[0], 128), lambda i, k: (0, i, 0)),
        ],
        out_shape=[
            jax.ShapeDtypeStruct((K, B // 128, 128), jnp.float32),
            jax.ShapeDtypeStruct((K, B // 128, 128), jnp.int32),
        ],
        scratch_shapes=[
            pltpu.VMEM((BQ, D), jnp.bfloat16),
            pltpu.VMEM((BQ, 128), jnp.float32),
            pltpu.VMEM((BQ, 128), jnp.float32),
            pltpu.VMEM((BQ, 128), jnp.float32),
            pltpu.VMEM((BQ, 128), jnp.int32),
            pltpu.VMEM((BQ, 128), jnp.int32),
            pltpu.VMEM((BQ, 128), jnp.int32),
        ],
    )(q, kn16)


_GATHER_WIN = 128


def _sc_gather(struct, idx_flat):
    n_idx = B * K

    @pl.kernel(
        out_type=jax.ShapeDtypeStruct((n_idx, S), jnp.float32),
        mesh=plsc.VectorSubcoreMesh(core_axis_name="core",
                                    subcore_axis_name="subcore"),
    )
    def _gather(x_hbm, i_hbm, o_hbm):
        def body(i_vmem, o_vmem):
            pltpu.sync_copy(x_hbm.at[i_vmem.at[0]], o_vmem)

        pltpu.emit_pipeline(
            body,
            grid=(n_idx // _GATHER_WIN,),
            in_specs=[pl.BlockSpec((1, _GATHER_WIN), lambda i: (0, i))],
            out_specs=[pl.BlockSpec((_GATHER_WIN, S), lambda i: (i, 0))],
            core_axis_name="subcore",
            dimension_semantics=(pltpu.PARALLEL,),
        )(i_hbm, o_hbm)

    return _gather(struct, idx_flat)


def _combine_kernel(w_ref, r0_ref, r1_ref, r2_ref, o_ref):
    w = w_ref[...]
    acc = (w[:, 0:1] * r0_ref[...] + w[:, 1:2] * r1_ref[...]
           + w[:, 2:3] * r2_ref[...])
    o_ref[...] = W_STRUCT * acc


def _combine(w, r0, r1, r2):
    bq = 1024
    return pl.pallas_call(
        _combine_kernel,
        grid=(B // bq,),
        in_specs=[
            pl.BlockSpec((bq, K), lambda i: (i, 0)),
            pl.BlockSpec((bq, S), lambda i: (i, 0)),
            pl.BlockSpec((bq, S), lambda i: (i, 0)),
            pl.BlockSpec((bq, S), lambda i: (i, 0)),
        ],
        out_specs=pl.BlockSpec((bq, S), lambda i: (i, 0)),
        out_shape=jax.ShapeDtypeStruct((B, S), jnp.float32),
    )(w, r0, r1, r2)


def kernel(sentence_embeddings, text_embeddings, structural_embeddings):
    kn16 = _normalize_keys(text_embeddings)
    wp, idxp = _topk(sentence_embeddings, kn16)
    idx_flat = idxp.reshape(1, K * B)
    retrieved = _sc_gather(structural_embeddings, idx_flat)
    r = retrieved.reshape(K, B, S)
    w = jnp.concatenate([wp[j].reshape(B, 1) for j in range(K)], axis=1)
    struct_ctx = _combine(w, r[0], r[1], r[2])
    return jnp.concatenate([sentence_embeddings, struct_ctx], axis=-1)
